# Initial kernel scaffold; baseline (speedup 1.0000x reference)
import jax
import jax.numpy as jnp
from jax.experimental import pallas as pl


def kernel(spikes, space_attn_mask, time_attn_mask, spacestamps, timestamps, embed_table, space_pos_table, ln_g, ln_b):
    raise NotImplementedError("write your pallas kernel here")


# speedup vs baseline: 34.3488x; 34.3488x over previous
import functools

import numpy as np
import jax
import jax.numpy as jnp
from jax import lax
from jax.experimental import pallas as pl
from jax.experimental.pallas import tpu as pltpu
from jax.experimental.pallas import tpu_sc as plsc

B = 64
T = 1024
C = 128
MULT = 4
HIDDEN = 512
MAX_SPIKES = 256
N_POS = 1024
SCALE = float(np.sqrt(HIDDEN))
LN_EPS = 1e-5

P = B * T
NC, NS, L = 2, 16, 16
NW = NC * NS
PPW = P // NW
CH = 64
NCH = PPW // CH

_REP = np.arange(16, dtype=np.int32) // 4
_OFF = np.arange(16, dtype=np.int32) % 4


def _ln_table(pos, g, b):
    def body(pos_ref, g_ref, b_ref, out_ref):
        x = pos_ref[...]
        mu = jnp.mean(x, axis=-1, keepdims=True)
        var = jnp.mean(jnp.square(x - mu), axis=-1, keepdims=True)
        out_ref[...] = (x - mu) / jnp.sqrt(var + LN_EPS) * g_ref[...] + b_ref[...]
    return pl.pallas_call(
        body,
        out_shape=jax.ShapeDtypeStruct((N_POS, HIDDEN), jnp.float32),
    )(pos, g.reshape(1, HIDDEN), b.reshape(1, HIDDEN))


def _make_sc_kernel():
    mesh = plsc.VectorSubcoreMesh(core_axis_name="c", subcore_axis_name="s")

    @functools.partial(
        pl.kernel,
        mesh=mesh,
        out_type=jax.ShapeDtypeStruct((P, HIDDEN), jnp.float32),
        compiler_params=pltpu.CompilerParams(needs_layout_passes=False),
        scratch_types=[
            pltpu.VMEM((MAX_SPIKES * MULT,), jnp.float32),
            pltpu.VMEM((CH * C,), jnp.int32),
            pltpu.VMEM((CH,), jnp.int32),
            pltpu.VMEM((CH, HIDDEN), jnp.float32),
            pltpu.SemaphoreType.DMA,
        ],
    )
    def sc_kernel(spk_hbm, st_hbm, lnp_hbm, sct_hbm, out_hbm,
                  sct_v, spk_v, st_v, pos_v, sem):
        wid = lax.axis_index("s") * NC + lax.axis_index("c")
        base0 = wid * PPW

        pltpu.sync_copy(sct_hbm, sct_v)
        for i in range(MAX_SPIKES * MULT // L):
            sct_v[pl.ds(i * L, L)] = sct_v[pl.ds(i * L, L)] * SCALE

        lanes = lax.iota(jnp.int32, L)
        rep = lax.shift_right_logical(lanes, 2)
        off = jnp.bitwise_and(lanes, 3)

        def chunk_body(ci, carry):
            base = base0 + ci * CH
            pltpu.sync_copy(spk_hbm.at[pl.ds(base * C, CH * C)], spk_v)
            pltpu.sync_copy(st_hbm.at[pl.ds(base, CH)], st_v)
            pltpu.async_copy(lnp_hbm.at[st_v], pos_v, sem).wait()

            def pair_body(p, c2):
                pbase = jnp.full((L,), p * C, dtype=jnp.int32) + rep
                for q in range(HIDDEN // L):
                    spk16 = plsc.load_gather(spk_v, [pbase + 4 * q])
                    fidx = spk16 * 4 + off
                    emb16 = plsc.load_gather(sct_v, [fidx])
                    pos_v[p, pl.ds(L * q, L)] = pos_v[p, pl.ds(L * q, L)] + emb16
                return c2

            lax.fori_loop(0, CH, pair_body, 0)
            pltpu.sync_copy(pos_v, out_hbm.at[pl.ds(base, CH)])
            return carry

        lax.fori_loop(0, NCH, chunk_body, 0)

    return sc_kernel


_SC_KERNEL = _make_sc_kernel()


def kernel(spikes, space_attn_mask, time_attn_mask, spacestamps, timestamps,
           embed_table, space_pos_table, ln_g, ln_b):
    lnp = _ln_table(space_pos_table, ln_g, ln_b)
    spikes2d = spikes.reshape(P * C)
    st = spacestamps.reshape(P)
    sct_flat = embed_table.reshape(MAX_SPIKES * MULT)
    x = _SC_KERNEL(spikes2d, st, lnp, sct_flat)
    x = x.reshape(B, T, HIDDEN)
    return (x, space_attn_mask, time_attn_mask, time_attn_mask, timestamps)


# speedup vs baseline: 40.2306x; 1.1712x over previous
import functools

import numpy as np
import jax
import jax.numpy as jnp
from jax import lax
from jax.experimental import pallas as pl
from jax.experimental.pallas import tpu as pltpu
from jax.experimental.pallas import tpu_sc as plsc

B = 64
T = 1024
C = 128
MULT = 4
HIDDEN = 512
MAX_SPIKES = 256
N_POS = 1024
SCALE = float(np.sqrt(HIDDEN))
LN_EPS = 1e-5

P = B * T
NC, NS, L = 2, 16, 16
NW = NC * NS
PPW = P // NW
CH = 64
NCH = PPW // CH


def _prep_tables(pos, g, b, emb):
    def body(pos_ref, g_ref, b_ref, emb_ref, lnp_ref, sct_ref):
        x = pos_ref[...]
        mu = jnp.mean(x, axis=-1, keepdims=True)
        var = jnp.mean(jnp.square(x - mu), axis=-1, keepdims=True)
        lnp_ref[...] = (x - mu) / jnp.sqrt(var + LN_EPS) * g_ref[...] + b_ref[...]
        sct_ref[...] = emb_ref[...] * SCALE
    return pl.pallas_call(
        body,
        out_shape=(
            jax.ShapeDtypeStruct((N_POS, HIDDEN), jnp.float32),
            jax.ShapeDtypeStruct((MAX_SPIKES, MULT), jnp.float32),
        ),
    )(pos, g.reshape(1, HIDDEN), b.reshape(1, HIDDEN), emb)


def _make_sc_kernel():
    mesh = plsc.VectorSubcoreMesh(core_axis_name="c", subcore_axis_name="s")

    @functools.partial(
        pl.kernel,
        mesh=mesh,
        out_type=jax.ShapeDtypeStruct((P, HIDDEN), jnp.float32),
        compiler_params=pltpu.CompilerParams(needs_layout_passes=False),
        scratch_types=[
            pltpu.VMEM((MAX_SPIKES * MULT,), jnp.float32),
            pltpu.VMEM((CH * C,), jnp.int32),
            pltpu.VMEM((CH * C,), jnp.int32),
            pltpu.VMEM((CH,), jnp.int32),
            pltpu.VMEM((CH,), jnp.int32),
            pltpu.VMEM((CH, HIDDEN), jnp.float32),
            pltpu.VMEM((CH, HIDDEN), jnp.float32),
            pltpu.SemaphoreType.DMA,
            pltpu.SemaphoreType.DMA,
            pltpu.SemaphoreType.DMA,
            pltpu.SemaphoreType.DMA,
            pltpu.SemaphoreType.DMA,
            pltpu.SemaphoreType.DMA,
        ],
    )
    def sc_kernel(spk_hbm, st_hbm, lnp_hbm, sct_hbm, out_hbm,
                  sct_v, spk_v0, spk_v1, st_v0, st_v1, pos_v0, pos_v1,
                  sem_i0, sem_i1, sem_g0, sem_g1, sem_o0, sem_o1):
        spk_v = (spk_v0, spk_v1)
        st_v = (st_v0, st_v1)
        pos_v = (pos_v0, pos_v1)
        sem_i = (sem_i0, sem_i1)
        sem_g = (sem_g0, sem_g1)
        sem_o = (sem_o0, sem_o1)

        wid = lax.axis_index("s") * NC + lax.axis_index("c")
        base0 = wid * PPW

        pltpu.sync_copy(sct_hbm, sct_v)

        lanes = lax.iota(jnp.int32, L)
        rep = lax.shift_right_logical(lanes, 2)
        off = jnp.bitwise_and(lanes, 3)

        def in_copy(ci, bi):
            base = base0 + ci * CH
            return (
                pltpu.make_async_copy(
                    spk_hbm.at[pl.ds(base * C, CH * C)], spk_v[bi], sem_i[bi]),
                pltpu.make_async_copy(
                    st_hbm.at[pl.ds(base, CH)], st_v[bi], sem_i[bi]),
            )

        def gather_copy(bi):
            return pltpu.make_async_copy(lnp_hbm.at[st_v[bi]], pos_v[bi], sem_g[bi])

        def out_copy(ci, bi):
            base = base0 + ci * CH
            return pltpu.make_async_copy(
                pos_v[bi], out_hbm.at[pl.ds(base, CH)], sem_o[bi])

        def compute(bi):
            def pair_body(p, c2):
                pbase = jnp.full((L,), p * C, dtype=jnp.int32) + rep
                pos_ref = pos_v[bi]
                for q in range(HIDDEN // L):
                    spk16 = plsc.load_gather(spk_v[bi], [pbase + 4 * q])
                    fidx = jnp.bitwise_or(lax.shift_left(spk16, 2), off)
                    emb16 = plsc.load_gather(sct_v, [fidx])
                    plsc.addupdate(pos_ref.at[p, pl.ds(L * q, L)], emb16)
                return c2
            lax.fori_loop(0, CH, pair_body, 0)

        for c_ in in_copy(0, 0):
            c_.start()
        for c_ in in_copy(0, 0):
            c_.wait()
        gather_copy(0).start()
        for c_ in in_copy(1, 1):
            c_.start()

        def process(ci, bi):
            nb = 1 - bi
            gather_copy(bi).wait()
            compute(bi)
            out_copy(ci, bi).start()

            @pl.when(ci + 1 < NCH)
            def _():
                for c_ in in_copy(ci + 1, nb):
                    c_.wait()

                @pl.when(ci >= 1)
                def _():
                    out_copy(ci - 1, nb).wait()

                gather_copy(nb).start()

                @pl.when(ci + 2 < NCH)
                def _():
                    for c_ in in_copy(ci + 2, bi):
                        c_.start()

        def loop_body(cj, carry):
            for b_ in range(2):
                process(2 * cj + b_, b_)
            return carry

        lax.fori_loop(0, NCH // 2, loop_body, 0)
        out_copy(NCH - 2, 0).wait()
        out_copy(NCH - 1, 1).wait()

    return sc_kernel


_SC_KERNEL = _make_sc_kernel()


def kernel(spikes, space_attn_mask, time_attn_mask, spacestamps, timestamps,
           embed_table, space_pos_table, ln_g, ln_b):
    lnp, sct = _prep_tables(space_pos_table, ln_g, ln_b, embed_table)
    spikes_flat = spikes.reshape(P * C)
    st = spacestamps.reshape(P)
    sct_flat = sct.reshape(MAX_SPIKES * MULT)
    x = _SC_KERNEL(spikes_flat, st, lnp, sct_flat)
    x = x.reshape(B, T, HIDDEN)
    return (x, space_attn_mask, time_attn_mask, time_attn_mask, timestamps)


# speedup vs baseline: 101.4958x; 2.5229x over previous
import functools

import numpy as np
import jax
import jax.numpy as jnp
from jax import lax
from jax.experimental import pallas as pl
from jax.experimental.pallas import tpu as pltpu
from jax.experimental.pallas import tpu_sc as plsc

B = 64
T = 1024
C = 128
MULT = 4
HIDDEN = 512
MAX_SPIKES = 256
N_POS = 1024
SCALE = float(np.sqrt(HIDDEN))
LN_EPS = 1e-5

P = B * T
NC, NS, L = 2, 16, 16
NW = NC * NS
PPW = P // NW
CH = 64
NCH = PPW // CH


def _prep_tables(pos, g, b, emb):
    def body(pos_ref, g_ref, b_ref, emb_ref, lnp_ref, sct_ref):
        x = pos_ref[...]
        mu = jnp.mean(x, axis=-1, keepdims=True)
        var = jnp.mean(jnp.square(x - mu), axis=-1, keepdims=True)
        lnp_ref[...] = (x - mu) / jnp.sqrt(var + LN_EPS) * g_ref[...] + b_ref[...]
        sct_ref[...] = emb_ref[...] * SCALE
    return pl.pallas_call(
        body,
        out_shape=(
            jax.ShapeDtypeStruct((N_POS, HIDDEN), jnp.float32),
            jax.ShapeDtypeStruct((MAX_SPIKES, MULT), jnp.float32),
        ),
    )(pos, g.reshape(1, HIDDEN), b.reshape(1, HIDDEN), emb)


def _make_sc_kernel():
    mesh = plsc.VectorSubcoreMesh(core_axis_name="c", subcore_axis_name="s")

    @functools.partial(
        pl.kernel,
        mesh=mesh,
        out_type=jax.ShapeDtypeStruct((P, HIDDEN), jnp.float32),
        compiler_params=pltpu.CompilerParams(needs_layout_passes=False),
        scratch_types=[
            pltpu.VMEM((MAX_SPIKES * MULT,), jnp.float32),
            pltpu.VMEM((CH * C,), jnp.int32),
            pltpu.VMEM((CH * C,), jnp.int32),
            pltpu.VMEM((CH,), jnp.int32),
            pltpu.VMEM((CH,), jnp.int32),
            pltpu.VMEM((CH, HIDDEN), jnp.float32),
            pltpu.VMEM((CH, HIDDEN), jnp.float32),
            pltpu.SemaphoreType.DMA,
            pltpu.SemaphoreType.DMA,
            pltpu.SemaphoreType.DMA,
            pltpu.SemaphoreType.DMA,
            pltpu.SemaphoreType.DMA,
            pltpu.SemaphoreType.DMA,
        ],
    )
    def sc_kernel(spk_hbm, st_hbm, lnp_hbm, sct_hbm, out_hbm,
                  sct_v, spk_v0, spk_v1, st_v0, st_v1, pos_v0, pos_v1,
                  sem_i0, sem_i1, sem_g0, sem_g1, sem_o0, sem_o1):
        spk_v = (spk_v0, spk_v1)
        st_v = (st_v0, st_v1)
        pos_v = (pos_v0, pos_v1)
        sem_i = (sem_i0, sem_i1)
        sem_g = (sem_g0, sem_g1)
        sem_o = (sem_o0, sem_o1)

        wid = lax.axis_index("s") * NC + lax.axis_index("c")
        base0 = wid * PPW

        pltpu.sync_copy(sct_hbm, sct_v)

        lanes = lax.iota(jnp.int32, L)
        rep = lax.shift_right_logical(lanes, 2)
        off = jnp.bitwise_and(lanes, 3)

        def in_copy(ci, bi):
            base = base0 + ci * CH
            return (
                pltpu.make_async_copy(
                    spk_hbm.at[pl.ds(base * C, CH * C)], spk_v[bi], sem_i[bi]),
                pltpu.make_async_copy(
                    st_hbm.at[pl.ds(base, CH)], st_v[bi], sem_i[bi]),
            )

        def gather_copy(bi):
            return pltpu.make_async_copy(lnp_hbm.at[st_v[bi]], pos_v[bi], sem_g[bi])

        def out_copy(ci, bi):
            base = base0 + ci * CH
            return pltpu.make_async_copy(
                pos_v[bi], out_hbm.at[pl.ds(base, CH)], sem_o[bi])

        def compute(bi):
            @plsc.parallel_loop(0, CH, unroll=2)
            def pair_body(p):
                pbase = jnp.full((L,), p * C, dtype=jnp.int32) + rep
                pos_ref = pos_v[bi]
                for q in range(HIDDEN // L):
                    spk16 = plsc.load_gather(spk_v[bi], [pbase + 4 * q])
                    fidx = jnp.bitwise_or(lax.shift_left(spk16, 2), off)
                    emb16 = plsc.load_gather(sct_v, [fidx])
                    plsc.addupdate(pos_ref.at[p, pl.ds(L * q, L)], emb16)

        for c_ in in_copy(0, 0):
            c_.start()
        for c_ in in_copy(0, 0):
            c_.wait()
        gather_copy(0).start()
        for c_ in in_copy(1, 1):
            c_.start()

        def process(ci, bi):
            nb = 1 - bi
            gather_copy(bi).wait()
            compute(bi)
            out_copy(ci, bi).start()

            @pl.when(ci + 1 < NCH)
            def _():
                for c_ in in_copy(ci + 1, nb):
                    c_.wait()

                @pl.when(ci >= 1)
                def _():
                    out_copy(ci - 1, nb).wait()

                gather_copy(nb).start()

                @pl.when(ci + 2 < NCH)
                def _():
                    for c_ in in_copy(ci + 2, bi):
                        c_.start()

        def loop_body(cj, carry):
            for b_ in range(2):
                process(2 * cj + b_, b_)
            return carry

        lax.fori_loop(0, NCH // 2, loop_body, 0)
        out_copy(NCH - 2, 0).wait()
        out_copy(NCH - 1, 1).wait()

    return sc_kernel


_SC_KERNEL = _make_sc_kernel()


def kernel(spikes, space_attn_mask, time_attn_mask, spacestamps, timestamps,
           embed_table, space_pos_table, ln_g, ln_b):
    lnp, sct = _prep_tables(space_pos_table, ln_g, ln_b, embed_table)
    spikes_flat = spikes.reshape(P * C)
    st = spacestamps.reshape(P)
    sct_flat = sct.reshape(MAX_SPIKES * MULT)
    x = _SC_KERNEL(spikes_flat, st, lnp, sct_flat)
    x = x.reshape(B, T, HIDDEN)
    return (x, space_attn_mask, time_attn_mask, time_attn_mask, timestamps)
